# Initial kernel scaffold; baseline (speedup 1.0000x reference)
import jax
import jax.numpy as jnp
from jax.experimental import pallas as pl


def kernel(mem_x, mem_y, mem_r, mem_g, mem_b, mem_a, px, py, pr, pg, pb, pa, idx):
    raise NotImplementedError("write your pallas kernel here")


# speedup vs baseline: 3.1180x; 3.1180x over previous
import functools

import jax
import jax.numpy as jnp
from jax import lax
from jax.experimental import pallas as pl
from jax.experimental.pallas import tpu as pltpu
from jax.experimental.pallas import tpu_sc as plsc

_M = 4194304
_B = 1048576
_CHUNK = 4096
_NW = 32
_NCHUNK = _M // _CHUNK
_PER_W = _NCHUNK // _NW
_NROWS = 6
_L = 16

_mesh = plsc.VectorSubcoreMesh(core_axis_name="c", subcore_axis_name="s")


@functools.partial(
    pl.kernel,
    out_type=jax.ShapeDtypeStruct((_NROWS, _M), jnp.float32),
    mesh=_mesh,
    scratch_types=[
        pltpu.VMEM((_CHUNK,), jnp.float32),
        pltpu.VMEM((_L,), jnp.int32),
        pltpu.SemaphoreType.DMA,
    ],
)
def _push_kernel(mem_x, mem_y, mem_r, mem_g, mem_b, mem_a,
                 px, py, pr, pg, pb, pa, idxv, out,
                 buf, idx_vm, sem):
    wid = lax.axis_index("s") * 2 + lax.axis_index("c")
    pltpu.sync_copy(idxv, idx_vm)
    idx = idx_vm[...][0]

    mems = (mem_x, mem_y, mem_r, mem_g, mem_b, mem_a)
    pts = (px, py, pr, pg, pb, pa)

    for r in range(_NROWS):
        def chunk_body(j, _, r=r):
            g0 = (wid + j * _NW) * _CHUNK
            inside = jnp.logical_and(g0 >= idx, g0 < idx + _B)
            g0 = pl.multiple_of(g0, _CHUNK)
            poff = pl.multiple_of(g0 - idx, _CHUNK)

            @pl.when(inside)
            def _():
                pltpu.async_copy(
                    pts[r].at[pl.ds(poff, _CHUNK)], buf, sem).wait()

            @pl.when(jnp.logical_not(inside))
            def _():
                pltpu.async_copy(
                    mems[r].at[pl.ds(g0, _CHUNK)], buf, sem).wait()

            if r >= 2:
                @pl.when(inside)
                def _():
                    def clamp_body(i, _):
                        v = buf[pl.ds(i * _L, _L)]
                        buf[pl.ds(i * _L, _L)] = jnp.minimum(
                            jnp.maximum(v, jnp.float32(0.001)),
                            jnp.float32(0.999))
                        return 0
                    lax.fori_loop(0, _CHUNK // _L, clamp_body, 0)

            pltpu.async_copy(buf, out.at[r, pl.ds(g0, _CHUNK)], sem).wait()
            return 0

        lax.fori_loop(0, _PER_W, chunk_body, 0)


def kernel(mem_x, mem_y, mem_r, mem_g, mem_b, mem_a,
           px, py, pr, pg, pb, pa, idx):
    idx32 = jnp.asarray(idx, dtype=jnp.int32)
    idxv = jnp.full((_L,), idx32, dtype=jnp.int32)
    out = _push_kernel(mem_x, mem_y, mem_r, mem_g, mem_b, mem_a,
                       px, py, pr, pg, pb, pa, idxv)
    return (idx32 + _B, out)


# speedup vs baseline: 5.8502x; 1.8763x over previous
import functools

import jax
import jax.numpy as jnp
from jax import lax
from jax.experimental import pallas as pl
from jax.experimental.pallas import tpu as pltpu
from jax.experimental.pallas import tpu_sc as plsc

_M = 4194304
_B = 1048576
_CHUNK = 4096
_NW = 32
_NCHUNK = _M // _CHUNK
_PER_W = _NCHUNK // _NW
_NROWS = 6
_L = 16
_G = 8
_GROUPS_PER_ROW = _PER_W // _G
_NGROUPS = _NROWS * _GROUPS_PER_ROW

_mesh = plsc.VectorSubcoreMesh(core_axis_name="c", subcore_axis_name="s")


@functools.partial(
    pl.kernel,
    out_type=jax.ShapeDtypeStruct((_NROWS, _M), jnp.float32),
    mesh=_mesh,
    scratch_types=[
        pltpu.VMEM((2, _G, _CHUNK), jnp.float32),
        pltpu.VMEM((_L,), jnp.int32),
        pltpu.SemaphoreType.DMA,
        pltpu.SemaphoreType.DMA,
        pltpu.SemaphoreType.DMA,
        pltpu.SemaphoreType.DMA,
    ],
)
def _push_kernel(mem_x, mem_y, mem_r, mem_g, mem_b, mem_a,
                 px, py, pr, pg, pb, pa, idxv, out,
                 buf, idx_vm, in_sem0, in_sem1, out_sem0, out_sem1):
    wid = lax.axis_index("s") * 2 + lax.axis_index("c")
    pltpu.sync_copy(idxv, idx_vm)
    idx = idx_vm[...][0]

    mems = (mem_x, mem_y, mem_r, mem_g, mem_b, mem_a)
    pts = (px, py, pr, pg, pb, pa)
    in_sems = (in_sem0, in_sem1)
    out_sems = (out_sem0, out_sem1)

    def chunk_start(gg, k):
        j = (gg % _GROUPS_PER_ROW) * _G + k
        g0 = (wid + j * _NW) * _CHUNK
        return pl.multiple_of(g0, _CHUNK)

    def issue_ins(gg, sel):
        r = gg // _GROUPS_PER_ROW

        def body(k, _):
            g0 = chunk_start(gg, k)
            inside = jnp.logical_and(g0 >= idx, g0 < idx + _B)
            poff = pl.multiple_of(g0 - idx, _CHUNK)

            @pl.when(inside)
            def _():
                pltpu.async_copy(pts[r].at[pl.ds(poff, _CHUNK)],
                                 buf.at[sel, k], in_sems[sel])

            @pl.when(jnp.logical_not(inside))
            def _():
                pltpu.async_copy(mems[r].at[pl.ds(g0, _CHUNK)],
                                 buf.at[sel, k], in_sems[sel])
            return 0

        lax.fori_loop(0, _G, body, 0)

    def drain(sem, sel):
        def body(k, _):
            pltpu.make_async_copy(mems[0].at[pl.ds(0, _CHUNK)],
                                  buf.at[sel, k], sem).wait()
            return 0

        lax.fori_loop(0, _G, body, 0)

    def clamp_group(gg, sel):
        def body(k, _):
            g0 = chunk_start(gg, k)
            inside = jnp.logical_and(g0 >= idx, g0 < idx + _B)

            @pl.when(inside)
            def _():
                def clamp_body(i, _):
                    o = i * (4 * _L)
                    for u in range(4):
                        v = buf[sel, k, pl.ds(o + u * _L, _L)]
                        buf[sel, k, pl.ds(o + u * _L, _L)] = jnp.minimum(
                            jnp.maximum(v, jnp.float32(0.001)),
                            jnp.float32(0.999))
                    return 0

                lax.fori_loop(0, _CHUNK // (4 * _L), clamp_body, 0)
            return 0

        lax.fori_loop(0, _G, body, 0)

    def issue_outs(gg, sel):
        r = gg // _GROUPS_PER_ROW

        def body(k, _):
            g0 = chunk_start(gg, k)
            pltpu.async_copy(buf.at[sel, k], out.at[r, pl.ds(g0, _CHUNK)],
                             out_sems[sel])
            return 0

        lax.fori_loop(0, _G, body, 0)

    issue_ins(0, 0)
    for gg in range(_NGROUPS):
        sel = gg % 2
        nsel = 1 - sel
        if gg + 1 < _NGROUPS:
            if gg >= 1:
                drain(out_sems[nsel], nsel)
            issue_ins(gg + 1, nsel)
        drain(in_sems[sel], sel)
        r = gg // _GROUPS_PER_ROW
        if r >= 2:
            clamp_group(gg, sel)
        issue_outs(gg, sel)
    drain(out_sems[(_NGROUPS - 2) % 2], (_NGROUPS - 2) % 2)
    drain(out_sems[(_NGROUPS - 1) % 2], (_NGROUPS - 1) % 2)


def kernel(mem_x, mem_y, mem_r, mem_g, mem_b, mem_a,
           px, py, pr, pg, pb, pa, idx):
    idx32 = jnp.asarray(idx, dtype=jnp.int32)
    idxv = jnp.full((_L,), idx32, dtype=jnp.int32)
    out = _push_kernel(mem_x, mem_y, mem_r, mem_g, mem_b, mem_a,
                       px, py, pr, pg, pb, pa, idxv)
    return (idx32 + _B, out)


# speedup vs baseline: 9.0429x; 1.5457x over previous
import functools

import jax
import jax.numpy as jnp
from jax import lax
from jax.experimental import pallas as pl
from jax.experimental.pallas import tpu as pltpu
from jax.experimental.pallas import tpu_sc as plsc

_M = 4194304
_B = 1048576
_CHUNK = 32768
_SUB = 4096
_NSUB = _CHUNK // _SUB
_NW = 32
_NCHUNK = _M // _CHUNK
_PER_W = _NCHUNK // _NW
_NROWS = 6
_L = 16
_NSETS = 3

_mesh = plsc.VectorSubcoreMesh(core_axis_name="c", subcore_axis_name="s")


@functools.partial(
    pl.kernel,
    out_type=jax.ShapeDtypeStruct((_NROWS, _M), jnp.float32),
    mesh=_mesh,
    scratch_types=[
        [pltpu.VMEM((_CHUNK,), jnp.float32)] * _NSETS,
        pltpu.VMEM((_L,), jnp.int32),
        [pltpu.SemaphoreType.DMA] * _NSETS,
        [pltpu.SemaphoreType.DMA] * _NSETS,
    ],
)
def _push_kernel(mem_x, mem_y, mem_r, mem_g, mem_b, mem_a,
                 px, py, pr, pg, pb, pa, idxv, out,
                 buf, idx_vm, in_sems, out_sems):
    wid = lax.axis_index("s") * 2 + lax.axis_index("c")
    pltpu.sync_copy(idxv, idx_vm)
    idx = idx_vm[...][0]

    mems = (mem_x, mem_y, mem_r, mem_g, mem_b, mem_a)
    pts = (px, py, pr, pg, pb, pa)

    def chunk_start(j):
        return pl.multiple_of((wid + j * _NW) * _CHUNK, _CHUNK)

    def chunk_conds(g0):
        fully_in = jnp.logical_and(g0 >= idx, g0 + _CHUNK <= idx + _B)
        fully_out = jnp.logical_or(g0 + _CHUNK <= idx, g0 >= idx + _B)
        strad = jnp.logical_not(jnp.logical_or(fully_in, fully_out))
        return fully_in, fully_out, strad

    def issue_ins(j, r, sel):
        g0 = chunk_start(j)
        fully_in, fully_out, strad = chunk_conds(g0)
        poff = pl.multiple_of(g0 - idx, _SUB)

        @pl.when(fully_in)
        def _():
            pltpu.async_copy(pts[r].at[pl.ds(poff, _CHUNK)],
                             buf[sel], in_sems[sel])

        @pl.when(fully_out)
        def _():
            pltpu.async_copy(mems[r].at[pl.ds(g0, _CHUNK)],
                             buf[sel], in_sems[sel])

        @pl.when(strad)
        def _():
            def sub(u, _):
                s0 = pl.multiple_of(g0 + u * _SUB, _SUB)
                ins = jnp.logical_and(s0 >= idx, s0 < idx + _B)
                soff = pl.multiple_of(s0 - idx, _SUB)
                boff = pl.multiple_of(u * _SUB, _SUB)
                dst = buf[sel].at[pl.ds(boff, _SUB)]

                @pl.when(ins)
                def _():
                    pltpu.async_copy(pts[r].at[pl.ds(soff, _SUB)],
                                     dst, in_sems[sel])

                @pl.when(jnp.logical_not(ins))
                def _():
                    pltpu.async_copy(mems[r].at[pl.ds(s0, _SUB)],
                                     dst, in_sems[sel])
                return 0

            lax.fori_loop(0, _NSUB, sub, 0)

    def drain_ins(j, sel):
        g0 = chunk_start(j)
        _, _, strad = chunk_conds(g0)

        @pl.when(jnp.logical_not(strad))
        def _():
            pltpu.make_async_copy(mems[0].at[pl.ds(0, _CHUNK)],
                                  buf[sel], in_sems[sel]).wait()

        @pl.when(strad)
        def _():
            def sub(u, _):
                pltpu.make_async_copy(
                    mems[0].at[pl.ds(0, _SUB)],
                    buf[sel].at[pl.ds(0, _SUB)], in_sems[sel]).wait()
                return 0

            lax.fori_loop(0, _NSUB, sub, 0)

    def drain_outs(sel):
        pltpu.make_async_copy(mems[0].at[pl.ds(0, _CHUNK)],
                              buf[sel], out_sems[sel]).wait()

    def clamp_stage(j, sel):
        g0 = chunk_start(j)

        def sub(u, _):
            s0 = g0 + u * _SUB
            ins = jnp.logical_and(s0 >= idx, s0 < idx + _B)

            @pl.when(ins)
            def _():
                def clamp_body(i, _):
                    o = u * _SUB + i * (4 * _L)
                    for v in range(4):
                        x = buf[sel][pl.ds(o + v * _L, _L)]
                        buf[sel][pl.ds(o + v * _L, _L)] = jnp.minimum(
                            jnp.maximum(x, jnp.float32(0.001)),
                            jnp.float32(0.999))
                    return 0

                lax.fori_loop(0, _SUB // (4 * _L), clamp_body, 0)
            return 0

        lax.fori_loop(0, _NSUB, sub, 0)

    def issue_outs(j, r, sel):
        g0 = chunk_start(j)
        pltpu.async_copy(buf[sel], out.at[r, pl.ds(g0, _CHUNK)],
                         out_sems[sel])

    issue_ins(jnp.int32(0), 0, 0)

    def outer(j, _):
        for r in range(_NROWS):
            sel = r % _NSETS
            nsel = (r + 1) % _NSETS

            if r >= 2:
                drain_outs(nsel)
            else:
                @pl.when(j >= 1)
                def _():
                    drain_outs(nsel)

            if r + 1 < _NROWS:
                issue_ins(j, r + 1, nsel)
            else:
                @pl.when(j + 1 < _PER_W)
                def _():
                    issue_ins(j + 1, 0, nsel)

            drain_ins(j, sel)
            if r >= 2:
                clamp_stage(j, sel)
            issue_outs(j, r, sel)
        return 0

    lax.fori_loop(0, _PER_W, outer, 0)

    drain_outs((_NROWS * _PER_W - 2) % _NSETS)
    drain_outs((_NROWS * _PER_W - 1) % _NSETS)


def kernel(mem_x, mem_y, mem_r, mem_g, mem_b, mem_a,
           px, py, pr, pg, pb, pa, idx):
    idx32 = jnp.asarray(idx, dtype=jnp.int32)
    idxv = jnp.full((_L,), idx32, dtype=jnp.int32)
    out = _push_kernel(mem_x, mem_y, mem_r, mem_g, mem_b, mem_a,
                       px, py, pr, pg, pb, pa, idxv)
    return (idx32 + _B, out)


# speedup vs baseline: 9.0716x; 1.0032x over previous
import functools

import jax
import jax.numpy as jnp
from jax import lax
from jax.experimental import pallas as pl
from jax.experimental.pallas import tpu as pltpu
from jax.experimental.pallas import tpu_sc as plsc

_M = 4194304
_B = 1048576
_CHUNK = 32768
_SUB = 4096
_NSUB = _CHUNK // _SUB
_NW = 32
_NCHUNK = _M // _CHUNK
_PER_W = _NCHUNK // _NW
_NROWS = 6
_L = 16
_NSETS = 3

_mesh = plsc.VectorSubcoreMesh(core_axis_name="c", subcore_axis_name="s")


@functools.partial(
    pl.kernel,
    out_type=jax.ShapeDtypeStruct((_NROWS, _M), jnp.float32),
    mesh=_mesh,
    scratch_types=[
        [pltpu.VMEM((_CHUNK,), jnp.float32)] * _NSETS,
        pltpu.VMEM((_L,), jnp.int32),
        [pltpu.SemaphoreType.DMA] * _NSETS,
        [pltpu.SemaphoreType.DMA] * _NSETS,
    ],
)
def _push_kernel(mem_x, mem_y, mem_r, mem_g, mem_b, mem_a,
                 px, py, pr, pg, pb, pa, idxv, out,
                 buf, idx_vm, in_sems, out_sems):
    wid = lax.axis_index("s") * 2 + lax.axis_index("c")
    pltpu.sync_copy(idxv, idx_vm)
    idx = idx_vm[...][0]

    mems = (mem_x, mem_y, mem_r, mem_g, mem_b, mem_a)
    pts = (px, py, pr, pg, pb, pa)

    def chunk_start(j):
        return pl.multiple_of((wid + j * _NW) * _CHUNK, _CHUNK)

    def chunk_conds(g0):
        fully_in = jnp.logical_and(g0 >= idx, g0 + _CHUNK <= idx + _B)
        fully_out = jnp.logical_or(g0 + _CHUNK <= idx, g0 >= idx + _B)
        strad = jnp.logical_not(jnp.logical_or(fully_in, fully_out))
        return fully_in, fully_out, strad

    def issue_ins(j, r, sel):
        g0 = chunk_start(j)
        fully_in, fully_out, strad = chunk_conds(g0)
        poff = pl.multiple_of(g0 - idx, _SUB)

        @pl.when(fully_in)
        def _():
            pltpu.async_copy(pts[r].at[pl.ds(poff, _CHUNK)],
                             buf[sel], in_sems[sel])

        @pl.when(fully_out)
        def _():
            pltpu.async_copy(mems[r].at[pl.ds(g0, _CHUNK)],
                             buf[sel], in_sems[sel])

        @pl.when(strad)
        def _():
            def sub(u, _):
                s0 = pl.multiple_of(g0 + u * _SUB, _SUB)
                ins = jnp.logical_and(s0 >= idx, s0 < idx + _B)
                soff = pl.multiple_of(s0 - idx, _SUB)
                boff = pl.multiple_of(u * _SUB, _SUB)
                dst = buf[sel].at[pl.ds(boff, _SUB)]

                @pl.when(ins)
                def _():
                    pltpu.async_copy(pts[r].at[pl.ds(soff, _SUB)],
                                     dst, in_sems[sel])

                @pl.when(jnp.logical_not(ins))
                def _():
                    pltpu.async_copy(mems[r].at[pl.ds(s0, _SUB)],
                                     dst, in_sems[sel])
                return 0

            lax.fori_loop(0, _NSUB, sub, 0)

    def drain_ins(j, sel):
        g0 = chunk_start(j)
        _, _, strad = chunk_conds(g0)

        @pl.when(jnp.logical_not(strad))
        def _():
            pltpu.make_async_copy(mems[0].at[pl.ds(0, _CHUNK)],
                                  buf[sel], in_sems[sel]).wait()

        @pl.when(strad)
        def _():
            def sub(u, _):
                pltpu.make_async_copy(
                    mems[0].at[pl.ds(0, _SUB)],
                    buf[sel].at[pl.ds(0, _SUB)], in_sems[sel]).wait()
                return 0

            lax.fori_loop(0, _NSUB, sub, 0)

    def drain_outs(sel):
        pltpu.make_async_copy(mems[0].at[pl.ds(0, _CHUNK)],
                              buf[sel], out_sems[sel]).wait()

    def clamp_stage(j, sel):
        g0 = chunk_start(j)

        def sub(u, _):
            s0 = g0 + u * _SUB
            ins = jnp.logical_and(s0 >= idx, s0 < idx + _B)

            base = pl.multiple_of(u * _SUB, _SUB)

            @pl.when(ins)
            def _():
                @plsc.parallel_loop(0, _SUB, _L, unroll=8)
                def _(i):
                    o = base + i
                    x = buf[sel][pl.ds(o, _L)]
                    buf[sel][pl.ds(o, _L)] = jnp.minimum(
                        jnp.maximum(x, jnp.float32(0.001)),
                        jnp.float32(0.999))
            return 0

        lax.fori_loop(0, _NSUB, sub, 0)

    def issue_outs(j, r, sel):
        g0 = chunk_start(j)
        pltpu.async_copy(buf[sel], out.at[r, pl.ds(g0, _CHUNK)],
                         out_sems[sel])

    issue_ins(jnp.int32(0), 0, 0)

    def outer(j, _):
        for r in range(_NROWS):
            sel = r % _NSETS
            nsel = (r + 1) % _NSETS

            if r >= 2:
                drain_outs(nsel)
            else:
                @pl.when(j >= 1)
                def _():
                    drain_outs(nsel)

            if r + 1 < _NROWS:
                issue_ins(j, r + 1, nsel)
            else:
                @pl.when(j + 1 < _PER_W)
                def _():
                    issue_ins(j + 1, 0, nsel)

            drain_ins(j, sel)
            if r >= 2:
                clamp_stage(j, sel)
            issue_outs(j, r, sel)
        return 0

    lax.fori_loop(0, _PER_W, outer, 0)

    drain_outs((_NROWS * _PER_W - 2) % _NSETS)
    drain_outs((_NROWS * _PER_W - 1) % _NSETS)


def kernel(mem_x, mem_y, mem_r, mem_g, mem_b, mem_a,
           px, py, pr, pg, pb, pa, idx):
    idx32 = jnp.asarray(idx, dtype=jnp.int32)
    idxv = jnp.full((_L,), idx32, dtype=jnp.int32)
    out = _push_kernel(mem_x, mem_y, mem_r, mem_g, mem_b, mem_a,
                       px, py, pr, pg, pb, pa, idxv)
    return (idx32 + _B, out)
